# baseline (speedup 1.0000x reference)
import numpy as np
import jax
import jax.numpy as jnp
from jax.experimental import pallas as pl
from jax.experimental.pallas import tpu as pltpu

N = 32768
M = N // 2
H = 128
N2 = 256
_TOPK = 8
_ROUNDS = 5
ROWS = 4


def _fft_topk_kernel(xe_ref, xo_ref, we_ref, wo_ref, f1r_ref, f1i_ref,
                     twr_ref, twi_ref, whr_ref, whi_ref, o_ref):
    hp = jax.lax.Precision.HIGHEST
    f32 = jnp.float32
    R = range(ROWS)

    def dot(a, b):
        return jax.lax.dot(a, b, precision=hp, preferred_element_type=f32)

    we, wo = we_ref[...], wo_ref[...]
    f1r, f1i = f1r_ref[...], f1i_ref[...]
    twr, twi = twr_ref[...], twi_ref[...]
    whr, whi = whr_ref[...], whi_ref[...]

    ye = [xe_ref[r] * we for r in R]
    yo = [xo_ref[r] * wo for r in R]

    nyq = [jnp.sum(ye[r] - yo[r]) for r in R]

    s1a = [dot(f1r, ye[r]) for r in R]
    s1b = [dot(f1i, yo[r]) for r in R]
    s1c = [dot(f1r, yo[r]) for r in R]
    s1d = [dot(f1i, ye[r]) for r in R]
    br = [s1a[r] - s1b[r] for r in R]
    bi = [s1c[r] + s1d[r] for r in R]
    br2 = [s1a[r] + s1b[r] for r in R]
    bi2 = [s1d[r] - s1c[r] for r in R]
    cr = [br[r] * twr - bi[r] * twi for r in R]
    ci = [br[r] * twi + bi[r] * twr for r in R]
    cr2 = [br2[r] * twr - bi2[r] * twi for r in R]
    ci2 = [br2[r] * twi + bi2[r] * twr for r in R]
    yr_ = [dot(cr[r], f1r) - dot(ci[r], f1i) for r in R]
    yi_ = [dot(cr[r], f1i) + dot(ci[r], f1r) for r in R]
    gr = [dot(cr2[r], f1r) - dot(ci2[r], f1i) for r in R]
    gi = [dot(cr2[r], f1i) + dot(ci2[r], f1r) for r in R]

    m2 = []
    for r in R:
        er = f32(0.5) * (yr_[r] + gr[r])
        ei = f32(0.5) * (yi_[r] + gi[r])
        o_r = f32(0.5) * (yi_[r] - gi[r])
        o_i = f32(-0.5) * (yr_[r] - gr[r])
        xr_ = er + whr * o_r - whi * o_i
        xi_ = ei + whr * o_i + whi * o_r
        m2.append(xr_ * xr_ + xi_ * xi_)

    row = jax.lax.broadcasted_iota(jnp.int32, (H, H), 0)
    col = jax.lax.broadcasted_iota(jnp.int32, (H, H), 1)
    tflat = row * H + col

    slots = [jnp.int32(_TOPK) for _ in R]
    nyq2 = [nyq[r] * nyq[r] for r in R]
    freqs = [[] for _ in R]
    coefs = [[] for _ in R]
    for _ in range(_ROUNDS):
        mx = [jnp.max(m2[r]) for r in R]
        p = [jnp.min(jnp.where(m2[r] == mx[r], tflat, jnp.int32(2 ** 30)))
             for r in R]
        for r in R:
            use_nyq = nyq2[r] > mx[r]
            k_tile = (p[r] >> 7) + ((p[r] & 127) << 7)
            mult = jnp.where(use_nyq | (p[r] == 0), jnp.int32(1),
                             jnp.int32(2))
            take = jnp.minimum(mult, slots[r])
            val2 = jnp.where(use_nyq, nyq2[r], mx[r])
            freqs[r].append(jnp.where(use_nyq, jnp.int32(M), k_tile))
            coefs[r].append(take.astype(f32) * jnp.sqrt(val2))
            slots[r] = slots[r] - take
            nyq2[r] = jnp.where(use_nyq, f32(-1.0), nyq2[r])
        m2 = [jnp.where((tflat == p[r]) & (freqs[r][-1] != M),
                        f32(-1.0), m2[r]) for r in R]

    K2 = 2 * _ROUNDS
    crow = jax.lax.broadcasted_iota(jnp.int32, (1, K2), 1)
    rrow = jax.lax.broadcasted_iota(jnp.int32, (K2, 1), 0)
    kvec = [jnp.zeros((1, K2), jnp.int32) for _ in R]
    cvec = [jnp.zeros((1, K2), f32) for _ in R]
    kcol = [jnp.zeros((K2, 1), jnp.int32) for _ in R]
    for j in range(_ROUNDS):
        csel = (crow >> 1) == j
        rsel = (rrow >> 1) == j
        for r in R:
            kvec[r] = jnp.where(csel, freqs[r][j], kvec[r])
            cvec[r] = jnp.where(csel, coefs[r][j], cvec[r])
            kcol[r] = jnp.where(rsel, freqs[r][j], kcol[r])
    rad = f32(2.0 * np.pi / N)
    i1v = jax.lax.broadcasted_iota(jnp.int32, (H, 1), 0)
    n2v = jax.lax.broadcasted_iota(jnp.int32, (1, N2), 1)
    ceven = (crow & 1) == 0
    reven = (rrow & 1) == 0
    ang_a = [(((i1v * N2) * kvec[r]) & (N - 1)).astype(f32) * rad for r in R]
    u = [jnp.where(ceven, cvec[r] * jnp.cos(ang_a[r]),
                   -cvec[r] * jnp.sin(ang_a[r])) for r in R]
    ang_b = [((kcol[r] * n2v) & (N - 1)).astype(f32) * rad for r in R]
    v = [jnp.where(reven, jnp.cos(ang_b[r]), jnp.sin(ang_b[r])) for r in R]
    for r in R:
        o_ref[r] = dot(u[r], v[r]) * f32(1.0 / N)


def _constants():
    n = np.arange(N)
    win = 0.5 * (1.0 - np.cos(2.0 * np.pi * n / N))
    we = win[0::2].reshape(H, H).astype(np.float32)
    wo = win[1::2].reshape(H, H).astype(np.float32)
    i1 = np.arange(H)
    f1 = np.exp(-2j * np.pi * np.outer(i1, i1) / H)
    tw = np.exp(-2j * np.pi * np.outer(i1, i1) / M)
    k = (i1[:, None] + H * i1[None, :])
    wh = np.exp(-2j * np.pi * k / N)
    return (we, wo,
            f1.real.astype(np.float32), f1.imag.astype(np.float32),
            tw.real.astype(np.float32), tw.imag.astype(np.float32),
            wh.real.astype(np.float32), wh.imag.astype(np.float32))


def kernel(inputs):
    x = inputs[:, :, 0]
    b = x.shape[0]
    x3 = x.reshape(b, H, N2)
    xe = x3[:, :, 0::2]
    xo = x3[:, :, 1::2]
    consts = _constants()
    out = pl.pallas_call(
        _fft_topk_kernel,
        grid=(b // ROWS,),
        in_specs=[pl.BlockSpec((ROWS, H, H), lambda i: (i, 0, 0)),
                  pl.BlockSpec((ROWS, H, H), lambda i: (i, 0, 0))]
                 + [pl.BlockSpec(c.shape, lambda i: (0, 0)) for c in consts],
        out_specs=pl.BlockSpec((ROWS, H, N2), lambda i: (i, 0, 0)),
        out_shape=jax.ShapeDtypeStruct((b, H, N2), jnp.float32),
        compiler_params=pltpu.CompilerParams(
            dimension_semantics=("parallel",)),
    )(xe, xo, *consts)
    return out.reshape(b, N)[:, :, None]
